# baseline (speedup 1.0000x reference)
import functools

import jax
import jax.numpy as jnp
from jax import lax
from jax.experimental import pallas as pl
from jax.experimental.pallas import tpu as pltpu
from jax.experimental.pallas import tpu_sc as plsc

_DIM = 64
_SCALE = 8.0
_LANES = 16
_NC, _NS = 2, 16
_NW = _NC * _NS
_CHUNK = 1600


@functools.partial(jax.jit, static_argnums=(2,))
def _gather_scaled(embedding, idx, B):
    b_per_w = B // _NW
    n_chunks = b_per_w // _CHUNK
    mesh = plsc.VectorSubcoreMesh(core_axis_name="c", subcore_axis_name="s")

    @functools.partial(
        pl.kernel,
        out_type=jax.ShapeDtypeStruct((B, _DIM), jnp.float32),
        mesh=mesh,
        scratch_types=[
            pltpu.VMEM((_CHUNK,), jnp.int32),
            pltpu.VMEM((_CHUNK, _DIM), jnp.float32),
            pltpu.SemaphoreType.DMA,
        ],
        compiler_params=pltpu.CompilerParams(use_tc_tiling_on_sc=False),
    )
    def k(emb_hbm, idx_hbm, out_hbm, idx_v, rows_v, sem):
        wid = lax.axis_index("s") * _NC + lax.axis_index("c")
        base = wid * b_per_w

        def chunk_body(c, carry):
            cb = base + c * _CHUNK
            pltpu.sync_copy(idx_hbm.at[pl.ds(cb, _CHUNK)], idx_v)
            pltpu.async_copy(emb_hbm.at[idx_v], rows_v, sem).wait()

            def row_body(r, carry2):
                for d in range(_DIM // _LANES):
                    sl = pl.ds(d * _LANES, _LANES)
                    rows_v[r, sl] = rows_v[r, sl] * _SCALE
                return carry2

            lax.fori_loop(0, _CHUNK, row_body, 0, unroll=4)
            pltpu.sync_copy(rows_v, out_hbm.at[pl.ds(cb, _CHUNK)])
            return carry

        lax.fori_loop(0, n_chunks, chunk_body, 0)

    return k(embedding, idx)


def kernel(x, embedding):
    B = x.shape[0] * x.shape[1]
    idx = x.reshape(B).astype(jnp.int32)
    out = _gather_scaled(embedding, idx, B)
    return out.reshape(x.shape + (embedding.shape[1],))


# speedup vs baseline: 1.0509x; 1.0509x over previous
import functools

import jax
import jax.numpy as jnp
from jax import lax
from jax.experimental import pallas as pl
from jax.experimental.pallas import tpu as pltpu
from jax.experimental.pallas import tpu_sc as plsc

_DIM = 64
_SCALE = 8.0
_LANES = 16
_NC, _NS = 2, 16
_NW = _NC * _NS
_CHUNK = 400
_NBUF = 4


def _gather_scaled(x, idx, embedding):
    B = x.shape[0] * x.shape[1]
    b_per_w = B // _NW
    n_chunks = b_per_w // _CHUNK
    n_groups = n_chunks // _NBUF
    out_shape = (x.shape[0], x.shape[1], _DIM)
    mesh = plsc.VectorSubcoreMesh(core_axis_name="c", subcore_axis_name="s")

    @functools.partial(
        pl.kernel,
        out_type=jax.ShapeDtypeStruct(out_shape, jnp.float32),
        mesh=mesh,
        scratch_types=[
            pltpu.VMEM((b_per_w,), jnp.int32),
            *[pltpu.VMEM((_CHUNK, _DIM), jnp.float32) for _ in range(_NBUF)],
            *[pltpu.SemaphoreType.DMA for _ in range(2 * _NBUF)],
        ],
        compiler_params=pltpu.CompilerParams(use_tc_tiling_on_sc=False),
    )
    def k(emb_hbm, idx_hbm, out3_hbm, idx_v, *bufs_and_sems):
        rows = bufs_and_sems[:_NBUF]
        gsem = bufs_and_sems[_NBUF:2 * _NBUF]
        wsem = bufs_and_sems[2 * _NBUF:3 * _NBUF]
        wid = lax.axis_index("s") * _NC + lax.axis_index("c")
        base = wid * b_per_w
        n1 = x.shape[1]
        r_per_chunk = _CHUNK // n1
        row0 = wid * (b_per_w // n1)
        pltpu.sync_copy(idx_hbm.at[pl.ds(base, b_per_w)], idx_v)

        def g_desc(c, b):
            return pltpu.make_async_copy(
                emb_hbm.at[idx_v.at[pl.ds(c * _CHUNK, _CHUNK)]], rows[b], gsem[b])

        def w_descs(c, b):
            return [
                pltpu.make_async_copy(
                    rows[b].at[pl.ds(i * n1, n1)],
                    out3_hbm.at[row0 + c * r_per_chunk + i],
                    wsem[b])
                for i in range(r_per_chunk)
            ]

        def w_start(c, b):
            for d in w_descs(c, b):
                d.start()

        def w_wait(c, b):
            for d in w_descs(c, b):
                d.wait()

        g_desc(0, 0).start()
        g_desc(1, 1).start()

        def group_body(g, carry):
            for b in range(_NBUF):
                c = g * _NBUF + b
                b2 = (b + 2) % _NBUF

                def prefetch():
                    w_wait(c - 2, b2)
                    g_desc(c + 2, b2).start()

                if b in (0, 1):
                    @pl.when(g > 0)
                    def _():
                        prefetch()

                    @pl.when(g == 0)
                    def _():
                        g_desc(c + 2, b2).start()
                else:
                    @pl.when(g < n_groups - 1)
                    def _():
                        prefetch()

                g_desc(c, b).wait()

                def row_body(r, carry2):
                    for d in range(_DIM // _LANES):
                        sl = pl.ds(d * _LANES, _LANES)
                        rows[b][r, sl] = rows[b][r, sl] * _SCALE
                    return carry2

                lax.fori_loop(0, _CHUNK, row_body, 0, unroll=4)
                w_start(c, b)
            return carry

        lax.fori_loop(0, n_groups, group_body, 0)
        for j in range(n_chunks - _NBUF, n_chunks):
            w_wait(j, j % _NBUF)

    return k(embedding, idx)


def kernel(x, embedding):
    idx = x.reshape(x.shape[0] * x.shape[1]).astype(jnp.int32)
    return _gather_scaled(x, idx, embedding)
